# Initial kernel scaffold; baseline (speedup 1.0000x reference)
import jax
import jax.numpy as jnp
from jax.experimental import pallas as pl


def kernel(x_nchw, weight_oihw, bias):
    raise NotImplementedError("write your pallas kernel here")


# speedup vs baseline: 1.1034x; 1.1034x over previous
import functools

import jax
import jax.numpy as jnp
from jax.experimental import pallas as pl
from jax.experimental.pallas import tpu as pltpu

N_CAPS = 4
D_FEAT = 8
GROUP = 4


def _make_body(nb):
    def body(p_ref, w_ref, b_ref, o_ref):
        for i in range(nb):
            p = p_ref[i]
            for c in range(N_CAPS):
                acc = jnp.dot(p, w_ref[c], preferred_element_type=jnp.float32)
                o_ref[i, c] = acc + b_ref[c]
    return body


@jax.jit
def _forward(x_nchw, weight_oihw, bias):
    N, Cin, H, W = x_nchw.shape
    Cout, wcin, KH, KW = weight_oihw.shape
    H_out = H - KH + 1
    W_out = W - KW + 1
    HW = H_out * W_out
    Kdim = KH * KW * Cin
    R = HW // GROUP

    x_nhwc = jnp.transpose(x_nchw, (0, 2, 3, 1)).astype(jnp.bfloat16)
    taps = []
    for kh in range(KH):
        for kw in range(KW):
            taps.append(x_nhwc[:, kh:kh + H_out, kw:kw + W_out, :])
    patches = jnp.concatenate(taps, axis=-1)
    p_big = patches.reshape(N, R, GROUP * Kdim)

    w2d = jnp.transpose(weight_oihw, (2, 3, 1, 0)).reshape(Kdim, Cout)
    base = w2d.reshape(Kdim, N_CAPS, D_FEAT).astype(jnp.float32)
    eye = jnp.eye(GROUP, dtype=jnp.float32)
    w_stack = jnp.einsum("jJ,kcd->cjkJd", eye, base)
    w_stack = w_stack.reshape(N_CAPS, GROUP * Kdim, GROUP * D_FEAT)
    w_stack = w_stack.astype(jnp.bfloat16)

    b2 = bias.astype(jnp.float32).reshape(N_CAPS, 1, 1, D_FEAT)
    b_stack = jnp.broadcast_to(b2, (N_CAPS, 1, GROUP, D_FEAT))
    b_stack = b_stack.reshape(N_CAPS, 1, GROUP * D_FEAT)

    nb = 4 if N % 4 == 0 else 1
    grid = (N // nb,)

    out = pl.pallas_call(
        _make_body(nb),
        out_shape=jax.ShapeDtypeStruct((N, N_CAPS, R, GROUP * D_FEAT), jnp.float32),
        grid=grid,
        in_specs=[
            pl.BlockSpec((nb, R, GROUP * Kdim), lambda i: (i, 0, 0)),
            pl.BlockSpec((N_CAPS, GROUP * Kdim, GROUP * D_FEAT), lambda i: (0, 0, 0)),
            pl.BlockSpec((N_CAPS, 1, GROUP * D_FEAT), lambda i: (0, 0, 0)),
        ],
        out_specs=pl.BlockSpec((nb, N_CAPS, R, GROUP * D_FEAT), lambda i: (i, 0, 0, 0)),
        compiler_params=pltpu.CompilerParams(dimension_semantics=("parallel",)),
    )(p_big, w_stack, b_stack)

    return out.reshape(N, N_CAPS * HW, D_FEAT).astype(x_nchw.dtype)


def kernel(x_nchw, weight_oihw, bias):
    return _forward(x_nchw, weight_oihw, bias)
